# baseline (speedup 1.0000x reference)
import jax
import jax.numpy as jnp
from jax import lax
from jax.experimental import pallas as pl

LATENT = 64
CODES = 1024
PIX = 1024
IMGS_PER_BLOCK = 2


def _vq_block(z_ref, cb_ref, zq_ref, idx_ref):
    cb = cb_ref[...]
    cb2 = cb * 2.0
    cb16 = cb.astype(jnp.bfloat16)
    cb_sq = jnp.sum(cb * cb, axis=1, keepdims=True)
    code_iota = lax.broadcasted_iota(jnp.int32, (CODES, PIX), 0)
    lane_iota = lax.broadcasted_iota(jnp.int32, (PIX, CODES), 1)
    for h in range(IMGS_PER_BLOCK):
        z = z_ref[LATENT * h : LATENT * (h + 1), :]
        z_sq = jnp.sum(z * z, axis=0, keepdims=True)
        scores = (z_sq + cb_sq) - lax.dot_general(
            cb2, z, (((1,), (0,)), ((), ())), preferred_element_type=jnp.float32
        )
        min_val = jnp.min(scores, axis=0, keepdims=True)
        idx_row = jnp.min(
            jnp.where(scores == min_val, code_iota, CODES), axis=0, keepdims=True
        )
        idx_ref[pl.ds(PIX * h, PIX)] = idx_row.reshape(PIX)
        idx_col = idx_row.reshape(PIX, 1)
        onehot = (lane_iota == idx_col).astype(jnp.bfloat16)
        zq_ref[pl.ds(PIX * h, PIX), :] = lax.dot_general(
            onehot, cb16, (((1,), (0,)), ((), ())),
            preferred_element_type=jnp.float32,
        )


def kernel(z_e, codebook):
    B, C, H, W = z_e.shape
    n_pix = B * H * W
    nb = B // IMGS_PER_BLOCK
    z2 = z_e.reshape(B * C, H * W)
    zq, idx = pl.pallas_call(
        _vq_block,
        grid=(nb,),
        in_specs=[
            pl.BlockSpec((IMGS_PER_BLOCK * C, H * W), lambda i: (i, 0)),
            pl.BlockSpec((CODES, LATENT), lambda i: (0, 0)),
        ],
        out_specs=[
            pl.BlockSpec((IMGS_PER_BLOCK * PIX, LATENT), lambda i: (i, 0)),
            pl.BlockSpec((IMGS_PER_BLOCK * PIX,), lambda i: (i,)),
        ],
        out_shape=[
            jax.ShapeDtypeStruct((n_pix, LATENT), jnp.float32),
            jax.ShapeDtypeStruct((n_pix,), jnp.int32),
        ],
    )(z2, codebook)
    return zq, idx


# speedup vs baseline: 1.5610x; 1.5610x over previous
import jax
import jax.numpy as jnp
from jax import lax
from jax.experimental import pallas as pl

LATENT = 64
CODES = 1024
PIX = 2048


def _vq_block(z_ref, cb_ref, zq_ref, idx_ref):
    z = jnp.concatenate([z_ref[0], z_ref[1]], axis=1)
    cb = cb_ref[...]
    cb_sq = jnp.sum(cb * cb, axis=1, keepdims=True)
    z_sq = jnp.sum(z * z, axis=0, keepdims=True)
    scores = (z_sq + cb_sq) - lax.dot_general(
        cb * 2.0, z, (((1,), (0,)), ((), ())), preferred_element_type=jnp.float32
    )
    min_val = jnp.min(scores, axis=0, keepdims=True)
    code_iota = lax.broadcasted_iota(jnp.int32, (CODES, PIX), 0)
    idx_row = jnp.min(
        jnp.where(scores == min_val, code_iota, CODES), axis=0, keepdims=True
    )
    idx_ref[...] = idx_row.reshape(PIX)
    idx_col = idx_row.reshape(PIX, 1)
    onehot = (
        lax.broadcasted_iota(jnp.int32, (PIX, CODES), 1) == idx_col
    ).astype(jnp.bfloat16)
    zq_ref[...] = lax.dot_general(
        onehot,
        cb.astype(jnp.bfloat16),
        (((1,), (0,)), ((), ())),
        preferred_element_type=jnp.float32,
    )


def kernel(z_e, codebook):
    B, C, H, W = z_e.shape
    n_pix = B * H * W
    nb = n_pix // PIX
    z3 = z_e.reshape(B, C, H * W)
    zq, idx = pl.pallas_call(
        _vq_block,
        grid=(nb,),
        in_specs=[
            pl.BlockSpec((2, LATENT, H * W), lambda i: (i, 0, 0)),
            pl.BlockSpec((CODES, LATENT), lambda i: (0, 0)),
        ],
        out_specs=[
            pl.BlockSpec((PIX, LATENT), lambda i: (i, 0)),
            pl.BlockSpec((PIX,), lambda i: (i,)),
        ],
        out_shape=[
            jax.ShapeDtypeStruct((n_pix, LATENT), jnp.float32),
            jax.ShapeDtypeStruct((n_pix,), jnp.int32),
        ],
    )(z3, codebook)
    return zq, idx
